# baseline (speedup 1.0000x reference)
import functools

import jax
import jax.numpy as jnp
from jax import lax
from jax.experimental import pallas as pl
from jax.experimental.pallas import tpu as pltpu
from jax.experimental.pallas import tpu_sc as plsc

_VOCAB = 100000
_D = 128
_NEG = 50
_NEG_PAD = 64
_ZP = 56
_B = 16384

_NW = 32
_BPW = _B // _NW
_CB = 8
_NCHUNK = _BPW // _CB
_ROWS = _CB * _NEG
_GCH = [(j * 128, 128) for j in range(3)] + [(384, 16)]
_NGRP = _NEG_PAD // 16
_SLAB = 16


_GATHER_DNUMS = lax.GatherDimensionNumbers(
    offset_dims=(), collapsed_slice_dims=(0,), start_index_map=(0,))


def _lane_rot(vec, perm):
    return lax.gather(vec, perm[:, None], _GATHER_DNUMS, slice_sizes=(1,),
                      mode=lax.GatherScatterMode.PROMISE_IN_BOUNDS)


def _body(t_ref, c_ref, ctab, ttab, out_ref,
          t_all, c_slab, ctx_v, tgt_v, z_v, sems):
    wid = lax.axis_index("s") * 2 + lax.axis_index("c")
    lane = lax.iota(jnp.int32, 16)

    pltpu.sync_copy(t_ref.at[pl.ds(wid * _BPW, _BPW)], t_all)

    def fetch_slab(ch):
        base = wid * _BPW * _NEG + (ch // _SLAB) * _SLAB * _ROWS
        pltpu.sync_copy(c_ref.at[pl.ds(base, _SLAB * _ROWS)], c_slab)

    def fire(ch, par):
        soff = (ch % _SLAB) * _ROWS
        pltpu.async_copy(ctab.at[t_all.at[pl.ds(ch * _CB, _CB)]],
                         ctx_v[par], sems[par])
        for off, sz in _GCH:
            pltpu.async_copy(
                ttab.at[c_slab.at[pl.ds(soff + off, sz)]],
                tgt_v[par].at[pl.ds(off, sz)], sems[par])

    def drain(ch, par):
        soff = (ch % _SLAB) * _ROWS
        pltpu.make_async_copy(ctab.at[t_all.at[pl.ds(ch * _CB, _CB)]],
                              ctx_v[par], sems[par]).wait()
        for off, sz in _GCH:
            pltpu.make_async_copy(
                ttab.at[c_slab.at[pl.ds(soff + off, sz)]],
                tgt_v[par].at[pl.ds(off, sz)], sems[par]).wait()

    def compute(ch, par):
        base = wid * _BPW + ch * _CB

        last = jnp.full((16,), 15, jnp.int32)

        def b_body(b, _):
            r0 = b * _NEG
            cvs = [ctx_v[par][b, pl.ds(kk * 16, 16)]
                   for kk in range(_D // 16)]
            zs = []
            for j in range(_NGRP):
                zv = jnp.full((16,), -jnp.inf, jnp.float32)
                for nn in range(min(16, _NEG - j * 16)):
                    n = j * 16 + nn
                    tv = [tgt_v[par][r0 + n, pl.ds(kk * 16, 16)]
                          for kk in range(_D // 16)]
                    a = [tv[i] * cvs[i] + tv[i + 4] * cvs[i + 4]
                         for i in range(4)]
                    acc = (a[0] + a[1]) + (a[2] + a[3])
                    tot = _lane_rot(lax.cumsum(acc, axis=0), last)
                    zv = jnp.where(lane == nn, tot, zv)
                zs.append(zv)
            m = jnp.max(jnp.maximum(jnp.maximum(zs[0], zs[1]),
                                    jnp.maximum(zs[2], zs[3])))
            es = [jnp.exp(zj - m) for zj in zs]
            s = jnp.sum(es[0] + es[1] + es[2] + es[3])
            for j in range(_NGRP):
                z_v[pl.ds(b * _ZP + j * 16, 16)] = es[j] / s
            return 0

        lax.fori_loop(0, _CB, b_body, 0)
        pltpu.sync_copy(z_v.at[pl.ds(0, _CB * _ZP)],
                        out_ref.at[pl.ds(base * _ZP, _CB * _ZP)])

    fetch_slab(0)
    fire(0, 0)

    def pair_body(i, _):
        ch0 = i * 2
        for par in range(2):
            ch = ch0 + par
            drain(ch, par)

            @pl.when(ch + 1 < _NCHUNK)
            def _():
                @pl.when((ch + 1) % _SLAB == 0)
                def _():
                    fetch_slab(ch + 1)

                fire(ch + 1, 1 - par)

            compute(ch, par)
        return 0

    lax.fori_loop(0, _NCHUNK // 2, pair_body, 0)


@jax.jit
def kernel(t, c, context_table, target_table):
    t_flat = t.reshape(_B)
    c_flat = c.reshape(_B * _NEG)
    k = functools.partial(
        pl.kernel,
        out_type=jax.ShapeDtypeStruct((_B * _ZP,), jnp.float32),
        mesh=plsc.VectorSubcoreMesh(core_axis_name="c", subcore_axis_name="s"),
        compiler_params=pltpu.CompilerParams(needs_layout_passes=False),
        scratch_types=[
            pltpu.VMEM((_BPW,), jnp.int32),
            pltpu.VMEM((_SLAB * _ROWS,), jnp.int32),
            [pltpu.VMEM((_CB, _D), jnp.float32) for _ in range(2)],
            [pltpu.VMEM((_ROWS, _D), jnp.float32) for _ in range(2)],
            pltpu.VMEM((_CB * _ZP + 8,), jnp.float32),
            [pltpu.SemaphoreType.DMA for _ in range(2)],
        ],
    )(_body)
    out = k(t_flat, c_flat, context_table, target_table)
    return out.reshape(_B, _ZP)[:, :_NEG]


# speedup vs baseline: 1.2244x; 1.2244x over previous
import functools

import jax
import jax.numpy as jnp
import numpy as np
from jax import lax
from jax.experimental import pallas as pl
from jax.experimental.pallas import tpu as pltpu
from jax.experimental.pallas import tpu_sc as plsc

_VOCAB = 100000
_D = 128
_NEG = 50
_NEG_PAD = 64
_ZP = 56
_B = 16384

_NW = 32
_BPW = _B // _NW
_CB = 8
_NCHUNK = _BPW // _CB
_ROWS = _CB * _NEG
_GCH = [(j * 128, 128) for j in range(3)] + [(384, 16)]
_NGRP = _NEG_PAD // 16
_SLAB = 16


_GATHER_DNUMS = lax.GatherDimensionNumbers(
    offset_dims=(), collapsed_slice_dims=(0,), start_index_map=(0,))


def _lane_rot(vec, perm):
    return lax.gather(vec, perm[:, None], _GATHER_DNUMS, slice_sizes=(1,),
                      mode=lax.GatherScatterMode.PROMISE_IN_BOUNDS)


def _tree16(vs, lane):
    lo8 = lane < 8
    l8 = lane - 8
    cur = list(vs)
    w = 16
    while len(cur) > 1:
        wp = w // 2
        xor = jnp.bitwise_xor(lane, wp)
        cxp = jnp.where(lo8, (lane // wp) * w + lane % wp, lane)
        cyp = jnp.where(lo8, lane, (l8 // wp) * w + l8 % wp)
        nxt = []
        for i in range(0, len(cur), 2):
            x, y = cur[i], cur[i + 1]
            sx = x + _lane_rot(x, xor)
            sy = y + _lane_rot(y, xor)
            cx = sx if w == 16 else _lane_rot(sx, cxp)
            cy = _lane_rot(sy, cyp)
            nxt.append(jnp.where(lo8, cx, cy))
        cur = nxt
        w = wp
    return cur[0]


def _body(t_ref, c_ref, ctab, ttab, out_ref,
          t_all, c_slab, ctx_v, tgt_v, z_v, sems):
    wid = lax.axis_index("s") * 2 + lax.axis_index("c")
    lane = lax.iota(jnp.int32, 16)

    pltpu.sync_copy(t_ref.at[pl.ds(wid * _BPW, _BPW)], t_all)

    def fetch_slab(ch):
        base = wid * _BPW * _NEG + (ch // _SLAB) * _SLAB * _ROWS
        pltpu.sync_copy(c_ref.at[pl.ds(base, _SLAB * _ROWS)], c_slab)

    def fire(ch, par):
        soff = (ch % _SLAB) * _ROWS
        pltpu.async_copy(ctab.at[t_all.at[pl.ds(ch * _CB, _CB)]],
                         ctx_v[par], sems[par])
        for off, sz in _GCH:
            pltpu.async_copy(
                ttab.at[c_slab.at[pl.ds(soff + off, sz)]],
                tgt_v[par].at[pl.ds(off, sz)], sems[par])

    def drain(ch, par):
        soff = (ch % _SLAB) * _ROWS
        pltpu.make_async_copy(ctab.at[t_all.at[pl.ds(ch * _CB, _CB)]],
                              ctx_v[par], sems[par]).wait()
        for off, sz in _GCH:
            pltpu.make_async_copy(
                ttab.at[c_slab.at[pl.ds(soff + off, sz)]],
                tgt_v[par].at[pl.ds(off, sz)], sems[par]).wait()

    def compute(ch, par):
        base = wid * _BPW + ch * _CB

        last = jnp.full((16,), 15, jnp.int32)

        def b_body(b, _):
            r0 = b * _NEG
            cvs = [ctx_v[par][b, pl.ds(kk * 16, 16)]
                   for kk in range(_D // 16)]
            def dot_acc(n):
                tv = [tgt_v[par][r0 + n, pl.ds(kk * 16, 16)]
                      for kk in range(_D // 16)]
                a = [tv[i] * cvs[i] + tv[i + 4] * cvs[i + 4]
                     for i in range(4)]
                return (a[0] + a[1]) + (a[2] + a[3])

            zs = []
            for j in range(_NGRP):
                if j * 16 + 16 <= _NEG:
                    zs.append(_tree16([dot_acc(j * 16 + nn)
                                       for nn in range(16)], lane))
                else:
                    zv = jnp.full((16,), -jnp.inf, jnp.float32)
                    for nn in range(_NEG - j * 16):
                        tot = _lane_rot(
                            lax.cumsum(dot_acc(j * 16 + nn), axis=0), last)
                        zv = jnp.where(lane == nn, tot, zv)
                    zs.append(zv)
            m = jnp.max(jnp.maximum(jnp.maximum(zs[0], zs[1]),
                                    jnp.maximum(zs[2], zs[3])))
            es = [jnp.exp(zj - m) for zj in zs]
            s = jnp.sum(es[0] + es[1] + es[2] + es[3])
            for j in range(_NGRP):
                z_v[pl.ds(b * _ZP + j * 16, 16)] = es[j] / s
            return 0

        lax.fori_loop(0, _CB, b_body, 0)
        pltpu.sync_copy(z_v.at[pl.ds(0, _CB * _ZP)],
                        out_ref.at[pl.ds(base * _ZP, _CB * _ZP)])

    fetch_slab(0)
    fire(0, 0)

    def pair_body(i, _):
        ch0 = i * 2
        for par in range(2):
            ch = ch0 + par
            drain(ch, par)

            @pl.when(ch + 1 < _NCHUNK)
            def _():
                @pl.when((ch + 1) % _SLAB == 0)
                def _():
                    fetch_slab(ch + 1)

                fire(ch + 1, 1 - par)

            compute(ch, par)
        return 0

    lax.fori_loop(0, _NCHUNK // 2, pair_body, 0)


@jax.jit
def kernel(t, c, context_table, target_table):
    t_flat = t.reshape(_B)
    c_flat = c.reshape(_B * _NEG)
    k = functools.partial(
        pl.kernel,
        out_type=jax.ShapeDtypeStruct((_B * _ZP,), jnp.float32),
        mesh=plsc.VectorSubcoreMesh(core_axis_name="c", subcore_axis_name="s"),
        compiler_params=pltpu.CompilerParams(needs_layout_passes=False),
        scratch_types=[
            pltpu.VMEM((_BPW,), jnp.int32),
            pltpu.VMEM((_SLAB * _ROWS,), jnp.int32),
            [pltpu.VMEM((_CB, _D), jnp.float32) for _ in range(2)],
            [pltpu.VMEM((_ROWS, _D), jnp.float32) for _ in range(2)],
            pltpu.VMEM((_CB * _ZP + 8,), jnp.float32),
            [pltpu.SemaphoreType.DMA for _ in range(2)],
        ],
    )(_body)
    out = k(t_flat, c_flat, context_table, target_table)
    return out.reshape(_B, _ZP)[:, :_NEG]
